# baseline (speedup 1.0000x reference)
import math

import jax
import jax.numpy as jnp
from jax import lax
from jax.experimental import pallas as pl
from jax.experimental.pallas import tpu as pltpu
from jax.experimental.pallas import tpu_sc as plsc

N = 10000
E = 320000
NP = 10240
NW = 32
WIN = NP // NW
ACC_R = WIN + 8
PKDUM = WIN
EPW = E // NW
CHUNK = 80
NCHUNK = EPW // CHUNK
SCAN = 160
NSC = E // SCAN
FIRE = 32
PCAP = 208
PSHIFT = PCAP - FIRE
HID = 256
ALPHA = 0.5
BE = 512
BN = 512


def _mesh():
    return plsc.VectorSubcoreMesh(core_axis_name="c", subcore_axis_name="s")


def _gather_edges_call(nf_p, src, dst):
    def body(nf_hbm, src_hbm, dst_hbm, hs_hbm, hd_hbm, srcv, dstv, nbuf, sem):
        wid = lax.axis_index("s") * 2 + lax.axis_index("c")

        def step(i, carry):
            base = wid * EPW + i * CHUNK
            pltpu.sync_copy(src_hbm.at[pl.ds(base, CHUNK)], srcv)
            pltpu.sync_copy(dst_hbm.at[pl.ds(base, CHUNK)], dstv)
            pltpu.async_copy(nf_hbm.at[srcv], nbuf, sem).wait()
            pltpu.sync_copy(nbuf, hs_hbm.at[pl.ds(base, CHUNK)])
            pltpu.async_copy(nf_hbm.at[dstv], nbuf, sem).wait()
            pltpu.sync_copy(nbuf, hd_hbm.at[pl.ds(base, CHUNK)])
            return carry

        lax.fori_loop(0, NCHUNK, step, 0)

    f = pl.kernel(
        body,
        mesh=_mesh(),
        out_type=(
            jax.ShapeDtypeStruct((E, 128), jnp.float32),
            jax.ShapeDtypeStruct((E, 128), jnp.float32),
        ),
        scratch_types=[
            pltpu.VMEM((CHUNK,), jnp.int32),
            pltpu.VMEM((CHUNK,), jnp.int32),
            pltpu.VMEM((CHUNK, 128), jnp.float32),
            pltpu.SemaphoreType.DMA,
        ],
    )
    return f(nf_p, src, dst)


def _scan_append(keyv, idsv, pend, wbase, tl):
    for g in range(SCAN // 16):
        d = keyv[pl.ds(g * 16, 16)]
        l = d - wbase
        ok = (l >= 0) & (l < WIN)
        vals = idsv[pl.ds(g * 16, 16)]
        pk = jnp.where(ok, vals * 512 + l, PKDUM)
        okn = jnp.where(ok, 1, 0)
        for lane in range(16):
            pend[pl.ds(tl, 16)] = lax.broadcast(pk[lane], (16,))
            tl = tl + okn[lane]
    return tl


def _drain_pad(pend, tl):
    dummy = lax.broadcast(jnp.int32(PKDUM), (16,))
    for j in range(FIRE // 16):
        pend[pl.ds(tl + j * 16, 16)] = dummy


def _seg_rows_call(table, ids, key, zacc):
    def body(table_hbm, ids_hbm, key_hbm, zacc_hbm,
             out_hbm,
             keyv, idsv, pend, gbuf, rows, acc, sem):
        c = lax.axis_index("c")
        s = lax.axis_index("s")
        wid = s * 2 + c
        wbase = wid * WIN
        pltpu.sync_copy(zacc_hbm, acc)

        def fire(tl):
            for g in range(FIRE // 16):
                pk = pend[pl.ds(g * 16, 16)]
                gbuf[pl.ds(g * 16, 16)] = lax.shift_right_logical(pk, 9)
            pltpu.async_copy(table_hbm.at[gbuf], rows, sem).wait()
            for g in range(FIRE // 16):
                lvec = pend[pl.ds(g * 16, 16)] & 511
                for lane in range(16):
                    lj = lvec[lane]
                    for k in range(HID // 16):
                        plsc.addupdate(acc.at[lj, pl.ds(k * 16, 16)],
                                       rows[g * 16 + lane, pl.ds(k * 16, 16)])
            for g in range(PSHIFT // 16):
                pend[pl.ds(g * 16, 16)] = pend[pl.ds(g * 16 + FIRE, 16)]
            return tl - FIRE

        def chunk(i, tl):
            ci = lax.rem(i + wid * (NSC // NW), NSC)
            base = ci * SCAN
            pltpu.sync_copy(key_hbm.at[pl.ds(base, SCAN)], keyv)
            pltpu.sync_copy(ids_hbm.at[pl.ds(base, SCAN)], idsv)
            tl = _scan_append(keyv, idsv, pend, wbase, tl)

            def maybe_fire(j, tl):
                return lax.cond(tl >= FIRE, fire, lambda t: t, tl)

            return lax.fori_loop(0, SCAN // FIRE, maybe_fire, tl)

        tl = lax.fori_loop(0, NSC, chunk, 0)
        _drain_pad(pend, tl)
        lax.cond(tl > 0, fire, lambda t: t, tl)
        pltpu.sync_copy(acc.at[pl.ds(0, WIN)], out_hbm.at[pl.ds(wbase, WIN)])

    f = pl.kernel(
        body,
        mesh=_mesh(),
        out_type=jax.ShapeDtypeStruct((NP, HID), jnp.float32),
        scratch_types=[
            pltpu.VMEM((SCAN,), jnp.int32),
            pltpu.VMEM((SCAN,), jnp.int32),
            pltpu.VMEM((PCAP,), jnp.int32),
            pltpu.VMEM((FIRE,), jnp.int32),
            pltpu.VMEM((FIRE, HID), jnp.float32),
            pltpu.VMEM((ACC_R, HID), jnp.float32),
            pltpu.SemaphoreType.DMA,
        ],
    )
    return f(table, ids, key, zacc)


def _seg_count_call(key, zdeg):
    def body(key_hbm, zdeg_hbm, deg_hbm, keyv, pend, dacc, sem):
        c = lax.axis_index("c")
        s = lax.axis_index("s")
        wid = s * 2 + c
        wbase = wid * WIN
        pltpu.sync_copy(zdeg_hbm, dacc)
        iota16 = lax.iota(jnp.int32, 16)
        onehot = jnp.where(iota16 == 0, 1.0, 0.0)

        def fire(tl):
            for g in range(FIRE // 16):
                lvec = pend[pl.ds(g * 16, 16)] & 511
                for lane in range(16):
                    plsc.addupdate(dacc.at[lvec[lane]], onehot)
            for g in range(PSHIFT // 16):
                pend[pl.ds(g * 16, 16)] = pend[pl.ds(g * 16 + FIRE, 16)]
            return tl - FIRE

        def chunk(i, tl):
            ci = lax.rem(i + wid * (NSC // NW), NSC)
            base = ci * SCAN
            pltpu.sync_copy(key_hbm.at[pl.ds(base, SCAN)], keyv)
            tl = _scan_append(keyv, keyv, pend, wbase, tl)

            def maybe_fire(j, tl):
                return lax.cond(tl >= FIRE, fire, lambda t: t, tl)

            return lax.fori_loop(0, SCAN // FIRE, maybe_fire, tl)

        tl = lax.fori_loop(0, NSC, chunk, 0)
        _drain_pad(pend, tl)
        lax.cond(tl > 0, fire, lambda t: t, tl)
        pltpu.sync_copy(dacc.at[pl.ds(0, WIN)], deg_hbm.at[pl.ds(wbase, WIN)])

    f = pl.kernel(
        body,
        mesh=_mesh(),
        out_type=jax.ShapeDtypeStruct((NP, 16), jnp.float32),
        scratch_types=[
            pltpu.VMEM((SCAN,), jnp.int32),
            pltpu.VMEM((PCAP,), jnp.int32),
            pltpu.VMEM((ACC_R, 16), jnp.float32),
            pltpu.SemaphoreType.DMA,
        ],
    )
    return f(key, zdeg)


def _silu(x):
    return x * (1.0 / (1.0 + jnp.exp(-x)))


def _edge_mlp_call(hs, hd, ef, Ws, Wd, We, wr, be0, We1, be1):
    def body(hs_r, hd_r, ef_r, Ws_r, Wd_r, We_r, wr_r, be0_r,
             We1_r, be1_r, out_r):
        xd = hs_r[:, 96:112] - hd_r[:, 96:112]
        rad = jnp.sum(xd * xd, axis=1, keepdims=True)
        z = (jnp.dot(hs_r[...], Ws_r[...], preferred_element_type=jnp.float32)
             + jnp.dot(hd_r[...], Wd_r[...], preferred_element_type=jnp.float32)
             + jnp.dot(ef_r[...], We_r[...], preferred_element_type=jnp.float32)
             + rad * wr_r[...] + be0_r[...])
        t = _silu(z)
        z2 = jnp.dot(t, We1_r[...], preferred_element_type=jnp.float32) + be1_r[...]
        out_r[...] = _silu(z2)

    full = lambda shape: pl.BlockSpec(shape, lambda i: (0, 0))
    return pl.pallas_call(
        body,
        grid=(E // BE,),
        in_specs=[
            pl.BlockSpec((BE, 128), lambda i: (i, 0)),
            pl.BlockSpec((BE, 128), lambda i: (i, 0)),
            pl.BlockSpec((BE, 8), lambda i: (i, 0)),
            full((128, HID)),
            full((128, HID)),
            full((8, HID)),
            full((1, HID)),
            full((1, HID)),
            full((HID, HID)),
            full((1, HID)),
        ],
        out_specs=pl.BlockSpec((BE, HID), lambda i: (i, 0)),
        out_shape=jax.ShapeDtypeStruct((E, HID), jnp.float32),
    )(hs, hd, ef, Ws, Wd, We, wr, be0, We1, be1)


def _node_mlp_call(nf_p, hn, deg_in, deg_out, Wa, Wb, bn0, Wn1, bn1):
    def body(nf_r, hn_r, din_r, dout_r, Wa_r, Wb_r, bn0_r, Wn1_r, bn1_r,
             h0_r, feat_r, nin_r, nout_r):
        z = (jnp.dot(nf_r[...], Wa_r[...], preferred_element_type=jnp.float32)
             + jnp.dot(hn_r[...], Wb_r[...], preferred_element_type=jnp.float32)
             + bn0_r[...])
        t = _silu(z)
        h = jnp.dot(t, Wn1_r[...], preferred_element_type=jnp.float32) + bn1_r[...]
        h0_r[...] = h
        nin_r[...] = lax.rsqrt(jnp.maximum(din_r[:, 0:1], 1.0))
        no = lax.rsqrt(jnp.maximum(dout_r[:, 0:1], 1.0))
        nout_r[...] = no
        feat_r[...] = h * no

    full = lambda shape: pl.BlockSpec(shape, lambda i: (0, 0))
    return pl.pallas_call(
        body,
        grid=(NP // BN,),
        in_specs=[
            pl.BlockSpec((BN, 128), lambda i: (i, 0)),
            pl.BlockSpec((BN, HID), lambda i: (i, 0)),
            pl.BlockSpec((BN, 16), lambda i: (i, 0)),
            pl.BlockSpec((BN, 16), lambda i: (i, 0)),
            full((128, HID)),
            full((HID, HID)),
            full((1, HID)),
            full((HID, HID)),
            full((1, HID)),
        ],
        out_specs=[
            pl.BlockSpec((BN, HID), lambda i: (i, 0)),
            pl.BlockSpec((BN, HID), lambda i: (i, 0)),
            pl.BlockSpec((BN, 1), lambda i: (i, 0)),
            pl.BlockSpec((BN, 1), lambda i: (i, 0)),
        ],
        out_shape=[
            jax.ShapeDtypeStruct((NP, HID), jnp.float32),
            jax.ShapeDtypeStruct((NP, HID), jnp.float32),
            jax.ShapeDtypeStruct((NP, 1), jnp.float32),
            jax.ShapeDtypeStruct((NP, 1), jnp.float32),
        ],
    )(nf_p, hn, deg_in, deg_out, Wa, Wb, bn0, Wn1, bn1)


def _gcn_layer_call(agg, res, nin, nout, Wg, bg, beta):
    def body(agg_r, res_r, nin_r, nout_r, Wg_r, bg_r, h_r, feat_r):
        rst = agg_r[...] * (nin_r[...] * (1.0 - ALPHA)) + ALPHA * res_r[...]
        y = ((1.0 - beta) * rst
             + beta * jnp.dot(rst, Wg_r[...], preferred_element_type=jnp.float32)
             + bg_r[...])
        h = _silu(y)
        h_r[...] = h
        feat_r[...] = h * nout_r[...]

    full = lambda shape: pl.BlockSpec(shape, lambda i: (0, 0))
    return pl.pallas_call(
        body,
        grid=(NP // BN,),
        in_specs=[
            pl.BlockSpec((BN, HID), lambda i: (i, 0)),
            pl.BlockSpec((BN, HID), lambda i: (i, 0)),
            pl.BlockSpec((BN, 1), lambda i: (i, 0)),
            pl.BlockSpec((BN, 1), lambda i: (i, 0)),
            full((HID, HID)),
            full((1, HID)),
        ],
        out_specs=[
            pl.BlockSpec((BN, HID), lambda i: (i, 0)),
            pl.BlockSpec((BN, HID), lambda i: (i, 0)),
        ],
        out_shape=[
            jax.ShapeDtypeStruct((NP, HID), jnp.float32),
            jax.ShapeDtypeStruct((NP, HID), jnp.float32),
        ],
    )(agg, res, nin, nout, Wg, bg)


def kernel(node_feat, coord, edge_feat, params, edge_index):
    f32 = jnp.float32
    src = edge_index[0]
    dst = edge_index[1]

    nf_p = (jnp.zeros((NP, 128), f32)
            .at[:N, :82].set(node_feat)
            .at[:N, 96:99].set(coord))
    ef_p = jnp.zeros((E, 8), f32).at[:, :6].set(edge_feat)

    We0 = params["We0"]
    Ws = jnp.zeros((128, HID), f32).at[:82].set(We0[:82])
    Wd = jnp.zeros((128, HID), f32).at[:82].set(We0[82:164])
    wr = We0[164:165]
    We = jnp.zeros((8, HID), f32).at[:6].set(We0[165:171])
    be0 = params["be0"][None, :]
    We1 = params["We1"]
    be1 = params["be1"][None, :]
    Wn0 = params["Wn0"]
    Wa = jnp.zeros((128, HID), f32).at[:82].set(Wn0[:82])
    Wb = Wn0[82:338]
    bn0 = params["bn0"][None, :]
    Wn1 = params["Wn1"]
    bn1 = params["bn1"][None, :]

    zacc = jnp.zeros((ACC_R, HID), f32)
    zdeg = jnp.zeros((ACC_R, 16), f32)
    iota_e = jnp.arange(E, dtype=jnp.int32)

    hs, hd = _gather_edges_call(nf_p, src, dst)
    msg = _edge_mlp_call(hs, hd, ef_p, Ws, Wd, We, wr, be0, We1, be1)
    hn = _seg_rows_call(msg, iota_e, dst, zacc)
    deg_in = _seg_count_call(dst, zdeg)
    deg_out = _seg_count_call(src, zdeg)
    h0, feat, nin, nout = _node_mlp_call(nf_p, hn, deg_in, deg_out,
                                         Wa, Wb, bn0, Wn1, bn1)

    h = h0
    for i in range(8):
        beta = math.log(1.0 / (i + 1) + 1.0)
        agg = _seg_rows_call(feat, src, dst, zacc)
        h, feat = _gcn_layer_call(agg, h0, nin, nout, params["Wg"][i],
                                  params["bg"][i][None, :], beta)
    return h[:N]
